# baseline (speedup 1.0000x reference)
import jax
import jax.numpy as jnp
from jax import lax
from jax.experimental import pallas as pl
from jax.experimental.pallas import tpu as pltpu
from jax.experimental.pallas import tpu_sc as plsc

_NC = 2
_NS = 16
_LANES = 16
_NW = _NC * _NS

_B, _L, _W, _H = 64, 80, 80, 40
_VOX_B = _L * _W * _H
_N = 16384
_NPTS = _B * _N

_PASSES = 8
_BPP = _B // _NC // _PASSES
_SPM = _BPP * _VOX_B
_PTS_PASS = _BPP * _N
_CHUNK = 2048
_CPP = _PTS_PASS // _NS // _CHUNK
_GROUPS = _CHUNK // _LANES
_NCORN = 8

_GRID_RES = 0.1
_INV_RES = 1.0 / _GRID_RES


def _floor_to_int(q):
  t = q.astype(jnp.int32)
  tf = t.astype(jnp.float32)
  adj = (tf > q)
  ti = jnp.where(adj, t - 1, t)
  return ti, jnp.where(adj, tf - 1.0, tf)


def _tec_body(vox_hbm, pts_hbm, hs_hbm, out_hbm,
              p_v, h_v, idx_v, w_v, val_v, part_v, sp_v, sem):
  c = lax.axis_index("c")
  s = lax.axis_index("s")
  lane = lax.iota(jnp.int32, _LANES)

  def do_chunk(off, lbase, acc):
    pltpu.sync_copy(pts_hbm.at[pl.ds(off * 3, _CHUNK * 3)], p_v)
    pltpu.sync_copy(hs_hbm.at[pl.ds(off, _CHUNK)], h_v)

    def group_body(i, _):
      sdx = i * _LANES
      lane3 = (sdx + lane) * 3
      px = plsc.load_gather(p_v, [lane3])
      py = plsc.load_gather(p_v, [lane3 + 1])
      pz = plsc.load_gather(p_v, [lane3 + 2])
      hh = h_v[pl.ds(sdx, _LANES)]

      x = px + 4.0
      y = py + 4.0
      z = pz + hh * 0.5

      xi, xf = _floor_to_int(x * _INV_RES)
      yi, yf = _floor_to_int(y * _INV_RES)
      zi, zf = _floor_to_int(z * _INV_RES)
      tx = (x - xf * _GRID_RES) * _INV_RES
      ty = (y - yf * _GRID_RES) * _INV_RES
      tz = (z - zf * _GRID_RES) * _INV_RES

      zero = jnp.zeros((_LANES,), jnp.int32)
      xmin = jnp.clip(xi, zero, _L - 1)
      xmax = jnp.clip(xi + 1, zero, _L - 1)
      ymin = jnp.clip(yi, zero, _W - 1)
      ymax = jnp.clip(yi + 1, zero, _W - 1)
      zmin = jnp.clip(zi, zero, _H - 1)
      zmax = jnp.clip(zi + 1, zero, _H - 1)

      axmin = lbase + xmin * (_W * _H)
      axmax = lbase + xmax * (_W * _H)
      bymin = ymin * _H
      bymax = ymax * _H

      ux = 1.0 - tx
      uy = 1.0 - ty
      uz = 1.0 - tz
      wxy_pp = tx * ty
      wxy_pm = tx * uy
      wxy_mp = ux * ty
      wxy_mm = ux * uy

      base_i = i * (_NCORN * _LANES)
      idx_v[pl.ds(base_i + 0 * _LANES, _LANES)] = axmax + bymax + zmax
      idx_v[pl.ds(base_i + 1 * _LANES, _LANES)] = axmax + bymax + zmin
      idx_v[pl.ds(base_i + 2 * _LANES, _LANES)] = axmax + bymin + zmax
      idx_v[pl.ds(base_i + 3 * _LANES, _LANES)] = axmax + bymin + zmin
      idx_v[pl.ds(base_i + 4 * _LANES, _LANES)] = axmin + bymax + zmax
      idx_v[pl.ds(base_i + 5 * _LANES, _LANES)] = axmin + bymax + zmin
      idx_v[pl.ds(base_i + 6 * _LANES, _LANES)] = axmin + bymin + zmax
      idx_v[pl.ds(base_i + 7 * _LANES, _LANES)] = axmin + bymin + zmin

      w_v[pl.ds(base_i + 0 * _LANES, _LANES)] = wxy_pp * tz
      w_v[pl.ds(base_i + 1 * _LANES, _LANES)] = wxy_pp * uz
      w_v[pl.ds(base_i + 2 * _LANES, _LANES)] = wxy_pm * tz
      w_v[pl.ds(base_i + 3 * _LANES, _LANES)] = wxy_pm * uz
      w_v[pl.ds(base_i + 4 * _LANES, _LANES)] = wxy_mp * tz
      w_v[pl.ds(base_i + 5 * _LANES, _LANES)] = wxy_mp * uz
      w_v[pl.ds(base_i + 6 * _LANES, _LANES)] = wxy_mm * tz
      w_v[pl.ds(base_i + 7 * _LANES, _LANES)] = wxy_mm * uz
      return _

    lax.fori_loop(0, _GROUPS, group_body, 0)

    pltpu.async_copy(sp_v.at[idx_v], val_v, sem).wait()

    def comb_body(i, acc_in):
      base_i = i * (_NCORN * _LANES)
      sdf = (val_v[pl.ds(base_i + 0 * _LANES, _LANES)]
             * w_v[pl.ds(base_i + 0 * _LANES, _LANES)])
      for cc in range(1, _NCORN):
        sdf = sdf + (val_v[pl.ds(base_i + cc * _LANES, _LANES)]
                     * w_v[pl.ds(base_i + cc * _LANES, _LANES)])
      ax = jnp.abs(sdf)
      hub = jnp.where(ax < 1.0, 0.5 * sdf * sdf, ax - 0.5)
      return acc_in + hub

    return lax.fori_loop(0, _GROUPS, comb_body, acc)

  acc = jnp.zeros((_LANES,), jnp.float32)
  for p in range(_PASSES):
    plsc.subcore_barrier()

    @pl.when(s == 0)
    def _stage():
      vb = c * (_SPM * _PASSES) + p * _SPM
      pltpu.sync_copy(vox_hbm.at[pl.ds(vb, _SPM)], sp_v)

    plsc.subcore_barrier()

    pass_pt = c * (_PTS_PASS * _PASSES) + p * _PTS_PASS
    for q in range(_CPP):
      toff = s * (_CPP * _CHUNK) + q * _CHUNK
      lbase = (toff // _N) * _VOX_B
      acc = do_chunk(pass_pt + toff, lbase, acc)

  part_v[...] = acc
  pltpu.sync_copy(part_v, out_hbm.at[s * _NC + c])


@jax.jit
def kernel(voxels, pts_centroid, height_gt):
  vox_flat = voxels.reshape(-1)
  pts_flat = pts_centroid.reshape(-1)
  hs = height_gt.reshape(-1)

  mesh = plsc.VectorSubcoreMesh(
      core_axis_name="c", subcore_axis_name="s",
      num_cores=_NC, num_subcores=_NS)
  kfn = pl.kernel(
      _tec_body,
      out_type=jax.ShapeDtypeStruct((_NW, _LANES), jnp.float32),
      mesh=mesh,
      scratch_types=[
          pltpu.VMEM((_CHUNK * 3,), jnp.float32),
          pltpu.VMEM((_CHUNK,), jnp.float32),
          pltpu.VMEM((_NCORN * _CHUNK,), jnp.int32),
          pltpu.VMEM((_NCORN * _CHUNK,), jnp.float32),
          pltpu.VMEM((_NCORN * _CHUNK,), jnp.float32),
          pltpu.VMEM((_LANES,), jnp.float32),
          pltpu.VMEM_SHARED((_SPM,), jnp.float32),
          pltpu.SemaphoreType.DMA,
      ],
      compiler_params=pltpu.CompilerParams(needs_layout_passes=False),
  )
  partials = kfn(vox_flat, pts_flat, hs)
  return jnp.sum(partials) / _NPTS


# speedup vs baseline: 7.6293x; 7.6293x over previous
import jax
import jax.numpy as jnp
from jax import lax
from jax.experimental import pallas as pl
from jax.experimental.pallas import tpu as pltpu
from jax.experimental.pallas import tpu_sc as plsc

_NC = 2
_NS = 16
_LANES = 16
_NW = _NC * _NS

_B, _L, _W, _H = 64, 80, 80, 40
_VOX_B = _L * _W * _H
_N = 16384
_NPTS = _B * _N

_PASSES = 8
_BPP = _B // _NC // _PASSES
_SPM = _BPP * _VOX_B
_PTS_PASS = _BPP * _N
_CHUNK = 2048
_CPP = _PTS_PASS // _NS // _CHUNK
_GROUPS = _CHUNK // _LANES
_NCORN = 8

_GRID_RES = 0.1
_INV_RES = 1.0 / _GRID_RES


def _floor_to_int(q):
  t = q.astype(jnp.int32)
  tf = t.astype(jnp.float32)
  adj = (tf > q)
  ti = jnp.where(adj, t - 1, t)
  return ti, jnp.where(adj, tf - 1.0, tf)


def _tec_body(vox_hbm, xs_hbm, ys_hbm, zs_hbm, hs_hbm, out_hbm,
              x_v, y_v, z_v, h_v, idx_v, w_v, val_v, part_v, sp_v, sem):
  c = lax.axis_index("c")
  s = lax.axis_index("s")

  def do_chunk(off, lbase, acc):
    pltpu.sync_copy(xs_hbm.at[pl.ds(off, _CHUNK)], x_v)
    pltpu.sync_copy(ys_hbm.at[pl.ds(off, _CHUNK)], y_v)
    pltpu.sync_copy(zs_hbm.at[pl.ds(off, _CHUNK)], z_v)
    pltpu.sync_copy(hs_hbm.at[pl.ds(off, _CHUNK)], h_v)

    def group_body(i, _):
      sdx = i * _LANES
      px = x_v[pl.ds(sdx, _LANES)]
      py = y_v[pl.ds(sdx, _LANES)]
      pz = z_v[pl.ds(sdx, _LANES)]
      hh = h_v[pl.ds(sdx, _LANES)]

      x = px + 4.0
      y = py + 4.0
      z = pz + hh * 0.5

      xi, xf = _floor_to_int(x * _INV_RES)
      yi, yf = _floor_to_int(y * _INV_RES)
      zi, zf = _floor_to_int(z * _INV_RES)
      tx = (x - xf * _GRID_RES) * _INV_RES
      ty = (y - yf * _GRID_RES) * _INV_RES
      tz = (z - zf * _GRID_RES) * _INV_RES

      zero = jnp.zeros((_LANES,), jnp.int32)
      xmin = jnp.clip(xi, zero, _L - 1)
      xmax = jnp.clip(xi + 1, zero, _L - 1)
      ymin = jnp.clip(yi, zero, _W - 1)
      ymax = jnp.clip(yi + 1, zero, _W - 1)
      zmin = jnp.clip(zi, zero, _H - 1)
      zmax = jnp.clip(zi + 1, zero, _H - 1)

      axmin = lbase + xmin * (_W * _H)
      axmax = lbase + xmax * (_W * _H)
      bzmin = zmin * _W
      bzmax = zmax * _W

      ux = 1.0 - tx
      uy = 1.0 - ty
      uz = 1.0 - tz
      wxy_pp = tx * ty
      wxy_pm = tx * uy
      wxy_mp = ux * ty
      wxy_mm = ux * uy

      base_i = i * (_NCORN * _LANES)
      idx_v[pl.ds(base_i + 0 * _LANES, _LANES)] = axmax + bzmax + ymax
      idx_v[pl.ds(base_i + 1 * _LANES, _LANES)] = axmax + bzmin + ymax
      idx_v[pl.ds(base_i + 2 * _LANES, _LANES)] = axmax + bzmax + ymin
      idx_v[pl.ds(base_i + 3 * _LANES, _LANES)] = axmax + bzmin + ymin
      idx_v[pl.ds(base_i + 4 * _LANES, _LANES)] = axmin + bzmax + ymax
      idx_v[pl.ds(base_i + 5 * _LANES, _LANES)] = axmin + bzmin + ymax
      idx_v[pl.ds(base_i + 6 * _LANES, _LANES)] = axmin + bzmax + ymin
      idx_v[pl.ds(base_i + 7 * _LANES, _LANES)] = axmin + bzmin + ymin

      w_v[pl.ds(base_i + 0 * _LANES, _LANES)] = wxy_pp * tz
      w_v[pl.ds(base_i + 1 * _LANES, _LANES)] = wxy_pp * uz
      w_v[pl.ds(base_i + 2 * _LANES, _LANES)] = wxy_pm * tz
      w_v[pl.ds(base_i + 3 * _LANES, _LANES)] = wxy_pm * uz
      w_v[pl.ds(base_i + 4 * _LANES, _LANES)] = wxy_mp * tz
      w_v[pl.ds(base_i + 5 * _LANES, _LANES)] = wxy_mp * uz
      w_v[pl.ds(base_i + 6 * _LANES, _LANES)] = wxy_mm * tz
      w_v[pl.ds(base_i + 7 * _LANES, _LANES)] = wxy_mm * uz
      return _

    lax.fori_loop(0, _GROUPS, group_body, 0)

    pltpu.async_copy(sp_v.at[idx_v], val_v, sem).wait()

    def comb_body(i, acc_in):
      base_i = i * (_NCORN * _LANES)
      sdf = (val_v[pl.ds(base_i + 0 * _LANES, _LANES)]
             * w_v[pl.ds(base_i + 0 * _LANES, _LANES)])
      for cc in range(1, _NCORN):
        sdf = sdf + (val_v[pl.ds(base_i + cc * _LANES, _LANES)]
                     * w_v[pl.ds(base_i + cc * _LANES, _LANES)])
      ax = jnp.abs(sdf)
      hub = jnp.where(ax < 1.0, 0.5 * sdf * sdf, ax - 0.5)
      return acc_in + hub

    return lax.fori_loop(0, _GROUPS, comb_body, acc)

  acc = jnp.zeros((_LANES,), jnp.float32)
  for p in range(_PASSES):
    plsc.subcore_barrier()

    @pl.when(s == 0)
    def _stage():
      vb = c * (_SPM * _PASSES) + p * _SPM
      pltpu.sync_copy(vox_hbm.at[pl.ds(vb, _SPM)], sp_v)

    plsc.subcore_barrier()

    pass_pt = c * (_PTS_PASS * _PASSES) + p * _PTS_PASS
    for q in range(_CPP):
      toff = s * (_CPP * _CHUNK) + q * _CHUNK
      lbase = (toff // _N) * _VOX_B
      acc = do_chunk(pass_pt + toff, lbase, acc)

  part_v[...] = acc
  pltpu.sync_copy(part_v, out_hbm.at[s * _NC + c])


@jax.jit
def kernel(voxels, pts_centroid, height_gt):
  vox_flat = voxels.transpose(0, 1, 3, 2).reshape(-1)
  xs = pts_centroid[..., 0].reshape(-1)
  ys = pts_centroid[..., 1].reshape(-1)
  zs = pts_centroid[..., 2].reshape(-1)
  hs = height_gt.reshape(-1)

  mesh = plsc.VectorSubcoreMesh(
      core_axis_name="c", subcore_axis_name="s",
      num_cores=_NC, num_subcores=_NS)
  kfn = pl.kernel(
      _tec_body,
      out_type=jax.ShapeDtypeStruct((_NW, _LANES), jnp.float32),
      mesh=mesh,
      scratch_types=[
          pltpu.VMEM((_CHUNK,), jnp.float32),
          pltpu.VMEM((_CHUNK,), jnp.float32),
          pltpu.VMEM((_CHUNK,), jnp.float32),
          pltpu.VMEM((_CHUNK,), jnp.float32),
          pltpu.VMEM((_NCORN * _CHUNK,), jnp.int32),
          pltpu.VMEM((_NCORN * _CHUNK,), jnp.float32),
          pltpu.VMEM((_NCORN * _CHUNK,), jnp.float32),
          pltpu.VMEM((_LANES,), jnp.float32),
          pltpu.VMEM_SHARED((_SPM,), jnp.float32),
          pltpu.SemaphoreType.DMA,
      ],
      compiler_params=pltpu.CompilerParams(needs_layout_passes=False),
  )
  partials = kfn(vox_flat, xs, ys, zs, hs)
  return jnp.sum(partials) / _NPTS


# speedup vs baseline: 7.9654x; 1.0440x over previous
import jax
import jax.numpy as jnp
from jax import lax
from jax.experimental import pallas as pl
from jax.experimental.pallas import tpu as pltpu
from jax.experimental.pallas import tpu_sc as plsc

_NC = 2
_NS = 16
_LANES = 16
_NW = _NC * _NS

_B, _L, _W, _H = 64, 80, 80, 40
_VOX_B = _L * _W * _H
_N = 16384
_NPTS = _B * _N

_STRIPES = _B // 8
_SPS = _STRIPES // _NC
_BPH = 4
_SPM = _BPH * _VOX_B
_COLS = 512
_PPC = _BPH * _COLS
_GROUPS = _PPC // _LANES
_GPR = _COLS // _LANES
_NCORN = 8

_GRID_RES = 0.1
_INV_RES = 1.0 / _GRID_RES


def _floor_to_int(q):
  t = q.astype(jnp.int32)
  tf = t.astype(jnp.float32)
  adj = (tf > q)
  ti = jnp.where(adj, t - 1, t)
  return ti, jnp.where(adj, tf - 1.0, tf)


def _tec_body(vox_hbm, xs_hbm, ys_hbm, zs_hbm, hs_hbm, out_hbm,
              x_v, y_v, z_v, h_v, idx_v, val_v,
              part_v, sp_v, sem):
  c = lax.axis_index("c")
  s = lax.axis_index("s")

  def do_subchunk(hh_, acc):

    def group_body(g, _):
      row = hh_ * _BPH + g // _GPR
      col = (g % _GPR) * _LANES
      lbase = (g // _GPR) * _VOX_B
      px = x_v[row, pl.ds(col, _LANES)]
      py = y_v[row, pl.ds(col, _LANES)]
      pz = z_v[row, pl.ds(col, _LANES)]
      hh = h_v[row, pl.ds(col, _LANES)]

      x = px + 4.0
      y = py + 4.0
      z = pz + hh * 0.5

      xi, xf = _floor_to_int(x * _INV_RES)
      yi, yf = _floor_to_int(y * _INV_RES)
      zi, zf = _floor_to_int(z * _INV_RES)
      tx = (x - xf * _GRID_RES) * _INV_RES
      ty = (y - yf * _GRID_RES) * _INV_RES
      tz = (z - zf * _GRID_RES) * _INV_RES

      zero = jnp.zeros((_LANES,), jnp.int32)
      xmin = jnp.clip(xi, zero, _L - 1)
      xmax = jnp.clip(xi + 1, zero, _L - 1)
      ymin = jnp.clip(yi, zero, _W - 1)
      ymax = jnp.clip(yi + 1, zero, _W - 1)
      zmin = jnp.clip(zi, zero, _H - 1)
      zmax = jnp.clip(zi + 1, zero, _H - 1)

      axmin = lbase + xmin * (_W * _H)
      axmax = lbase + xmax * (_W * _H)
      bzmin = zmin * _W
      bzmax = zmax * _W

      base_i = g * (_NCORN * _LANES)
      idx_v[pl.ds(base_i + 0 * _LANES, _LANES)] = axmax + bzmax + ymax
      idx_v[pl.ds(base_i + 1 * _LANES, _LANES)] = axmax + bzmin + ymax
      idx_v[pl.ds(base_i + 2 * _LANES, _LANES)] = axmax + bzmax + ymin
      idx_v[pl.ds(base_i + 3 * _LANES, _LANES)] = axmax + bzmin + ymin
      idx_v[pl.ds(base_i + 4 * _LANES, _LANES)] = axmin + bzmax + ymax
      idx_v[pl.ds(base_i + 5 * _LANES, _LANES)] = axmin + bzmin + ymax
      idx_v[pl.ds(base_i + 6 * _LANES, _LANES)] = axmin + bzmax + ymin
      idx_v[pl.ds(base_i + 7 * _LANES, _LANES)] = axmin + bzmin + ymin

      x_v[row, pl.ds(col, _LANES)] = tx
      y_v[row, pl.ds(col, _LANES)] = ty
      z_v[row, pl.ds(col, _LANES)] = tz
      return _

    lax.fori_loop(0, _GROUPS, group_body, 0)

    pltpu.async_copy(sp_v.at[idx_v], val_v, sem).wait()

    def comb_body(g, acc_in):
      row = hh_ * _BPH + g // _GPR
      col = (g % _GPR) * _LANES
      tx = x_v[row, pl.ds(col, _LANES)]
      ty = y_v[row, pl.ds(col, _LANES)]
      tz = z_v[row, pl.ds(col, _LANES)]
      ux = 1.0 - tx
      uy = 1.0 - ty
      uz = 1.0 - tz
      wxy_pp = tx * ty
      wxy_pm = tx * uy
      wxy_mp = ux * ty
      wxy_mm = ux * uy
      base_i = g * (_NCORN * _LANES)
      sdf = val_v[pl.ds(base_i + 0 * _LANES, _LANES)] * (wxy_pp * tz)
      sdf = sdf + val_v[pl.ds(base_i + 1 * _LANES, _LANES)] * (wxy_pp * uz)
      sdf = sdf + val_v[pl.ds(base_i + 2 * _LANES, _LANES)] * (wxy_pm * tz)
      sdf = sdf + val_v[pl.ds(base_i + 3 * _LANES, _LANES)] * (wxy_pm * uz)
      sdf = sdf + val_v[pl.ds(base_i + 4 * _LANES, _LANES)] * (wxy_mp * tz)
      sdf = sdf + val_v[pl.ds(base_i + 5 * _LANES, _LANES)] * (wxy_mp * uz)
      sdf = sdf + val_v[pl.ds(base_i + 6 * _LANES, _LANES)] * (wxy_mm * tz)
      sdf = sdf + val_v[pl.ds(base_i + 7 * _LANES, _LANES)] * (wxy_mm * uz)
      ax = jnp.abs(sdf)
      hub = jnp.where(ax < 1.0, 0.5 * sdf * sdf, ax - 0.5)
      return acc_in + hub

    return lax.fori_loop(0, _GROUPS, comb_body, acc)

  acc = jnp.zeros((_LANES,), jnp.float32)
  for sp in range(_SPS):
    k = c * _SPS + sp
    row0 = k * 8
    for h in range(2):
      plsc.subcore_barrier()

      @pl.when(s == 0)
      def _stage():
        vb = (k * 8 + h * _BPH) * _VOX_B
        pltpu.sync_copy(vox_hbm.at[pl.ds(vb, _SPM)], sp_v)

      plsc.subcore_barrier()
      for u in range(2):
        col0 = s * (2 * _COLS) + u * _COLS
        pltpu.sync_copy(xs_hbm.at[pl.ds(row0, 8), pl.ds(col0, _COLS)], x_v)
        pltpu.sync_copy(ys_hbm.at[pl.ds(row0, 8), pl.ds(col0, _COLS)], y_v)
        pltpu.sync_copy(zs_hbm.at[pl.ds(row0, 8), pl.ds(col0, _COLS)], z_v)
        pltpu.sync_copy(hs_hbm.at[pl.ds(row0, 8), pl.ds(col0, _COLS)], h_v)
        acc = do_subchunk(h, acc)

  part_v[...] = acc
  pltpu.sync_copy(part_v, out_hbm.at[s * _NC + c])


@jax.jit
def kernel(voxels, pts_centroid, height_gt):
  vox_flat = voxels.transpose(0, 1, 3, 2).reshape(-1)
  xs = pts_centroid[..., 0]
  ys = pts_centroid[..., 1]
  zs = pts_centroid[..., 2]
  hs = height_gt

  mesh = plsc.VectorSubcoreMesh(
      core_axis_name="c", subcore_axis_name="s",
      num_cores=_NC, num_subcores=_NS)
  kfn = pl.kernel(
      _tec_body,
      out_type=jax.ShapeDtypeStruct((_NW, _LANES), jnp.float32),
      mesh=mesh,
      scratch_types=[
          pltpu.VMEM((8, _COLS), jnp.float32),
          pltpu.VMEM((8, _COLS), jnp.float32),
          pltpu.VMEM((8, _COLS), jnp.float32),
          pltpu.VMEM((8, _COLS), jnp.float32),
          pltpu.VMEM((_NCORN * _PPC,), jnp.int32),
          pltpu.VMEM((_NCORN * _PPC,), jnp.float32),
          pltpu.VMEM((_LANES,), jnp.float32),
          pltpu.VMEM_SHARED((_SPM,), jnp.float32),
          pltpu.SemaphoreType.DMA,
      ],
      compiler_params=pltpu.CompilerParams(needs_layout_passes=False),
  )
  partials = kfn(vox_flat, xs, ys, zs, hs)
  return jnp.sum(partials) / _NPTS


# speedup vs baseline: 9.4660x; 1.1884x over previous
import jax
import jax.numpy as jnp
from jax import lax
from jax.experimental import pallas as pl
from jax.experimental.pallas import tpu as pltpu
from jax.experimental.pallas import tpu_sc as plsc

_NC = 2
_NS = 16
_LANES = 16
_NW = _NC * _NS

_B, _L, _W, _H = 64, 80, 80, 40
_VOX_B = _L * _W * _H
_N = 16384
_NPTS = _B * _N

_SPS = 4
_BPH = 4
_SPM = _BPH * _VOX_B
_COLS = 256
_NSUB = 4
_PPC = _BPH * _COLS
_GROUPS = _PPC // _LANES
_GPR = _COLS // _LANES
_NCORN = 8

_GRID_RES = 0.1
_INV_RES = 1.0 / _GRID_RES


def _floor_to_int(q):
  t = q.astype(jnp.int32)
  tf = t.astype(jnp.float32)
  adj = (tf > q)
  ti = jnp.where(adj, t - 1, t)
  return ti, jnp.where(adj, tf - 1.0, tf)


def _tec_body(vox_hbm, xs_hbm, ys_hbm, zs_hbm, hs_hbm, out_hbm,
              x_v, y_v, z_v, h_v, idx_v0, idx_v1, w_v, val_v0, val_v1,
              part_v, sp_v, gsems, psems):
  c = lax.axis_index("c")
  s = lax.axis_index("s")
  idx_vs = (idx_v0, idx_v1)
  val_vs = (val_v0, val_v1)

  def fire_pts(row0, u, d):
    col0 = s * (_NSUB * _COLS) + u * _COLS
    return [
        pltpu.async_copy(src.at[pl.ds(row0, 8), pl.ds(col0, _COLS)],
                         dst.at[d], psems.at[d])
        for src, dst in ((xs_hbm, x_v), (ys_hbm, y_v),
                         (zs_hbm, z_v), (hs_hbm, h_v))
    ]

  def compute(hh_, d):

    def group_body(g, _):
      row = hh_ * _BPH + g // _GPR
      col = (g % _GPR) * _LANES
      lbase = (g // _GPR) * _VOX_B
      px = x_v[d, row, pl.ds(col, _LANES)]
      py = y_v[d, row, pl.ds(col, _LANES)]
      pz = z_v[d, row, pl.ds(col, _LANES)]
      hh = h_v[d, row, pl.ds(col, _LANES)]

      x = px + 4.0
      y = py + 4.0
      z = pz + hh * 0.5

      xi, xf = _floor_to_int(x * _INV_RES)
      yi, yf = _floor_to_int(y * _INV_RES)
      zi, zf = _floor_to_int(z * _INV_RES)
      tx = (x - xf * _GRID_RES) * _INV_RES
      ty = (y - yf * _GRID_RES) * _INV_RES
      tz = (z - zf * _GRID_RES) * _INV_RES

      zero = jnp.zeros((_LANES,), jnp.int32)
      xmin = jnp.clip(xi, zero, _L - 1)
      xmax = jnp.clip(xi + 1, zero, _L - 1)
      ymin = jnp.clip(yi, zero, _W - 1)
      ymax = jnp.clip(yi + 1, zero, _W - 1)
      zmin = jnp.clip(zi, zero, _H - 1)
      zmax = jnp.clip(zi + 1, zero, _H - 1)

      axmin = lbase + xmin * (_W * _H)
      axmax = lbase + xmax * (_W * _H)
      bzmin = zmin * _W
      bzmax = zmax * _W

      ux = 1.0 - tx
      uy = 1.0 - ty
      uz = 1.0 - tz
      wxy_pp = tx * ty
      wxy_pm = tx * uy
      wxy_mp = ux * ty
      wxy_mm = ux * uy

      base_i = g * (_NCORN * _LANES)
      idx_d = idx_vs[d]
      idx_d[pl.ds(base_i + 0 * _LANES, _LANES)] = axmax + bzmax + ymax
      idx_d[pl.ds(base_i + 1 * _LANES, _LANES)] = axmax + bzmin + ymax
      idx_d[pl.ds(base_i + 2 * _LANES, _LANES)] = axmax + bzmax + ymin
      idx_d[pl.ds(base_i + 3 * _LANES, _LANES)] = axmax + bzmin + ymin
      idx_d[pl.ds(base_i + 4 * _LANES, _LANES)] = axmin + bzmax + ymax
      idx_d[pl.ds(base_i + 5 * _LANES, _LANES)] = axmin + bzmin + ymax
      idx_d[pl.ds(base_i + 6 * _LANES, _LANES)] = axmin + bzmax + ymin
      idx_d[pl.ds(base_i + 7 * _LANES, _LANES)] = axmin + bzmin + ymin

      w_v[d, pl.ds(base_i + 0 * _LANES, _LANES)] = wxy_pp * tz
      w_v[d, pl.ds(base_i + 1 * _LANES, _LANES)] = wxy_pp * uz
      w_v[d, pl.ds(base_i + 2 * _LANES, _LANES)] = wxy_pm * tz
      w_v[d, pl.ds(base_i + 3 * _LANES, _LANES)] = wxy_pm * uz
      w_v[d, pl.ds(base_i + 4 * _LANES, _LANES)] = wxy_mp * tz
      w_v[d, pl.ds(base_i + 5 * _LANES, _LANES)] = wxy_mp * uz
      w_v[d, pl.ds(base_i + 6 * _LANES, _LANES)] = wxy_mm * tz
      w_v[d, pl.ds(base_i + 7 * _LANES, _LANES)] = wxy_mm * uz
      return _

    lax.fori_loop(0, _GROUPS, group_body, 0)

  def fire_gather(d):
    return pltpu.async_copy(sp_v.at[idx_vs[d]], val_vs[d], gsems.at[d])

  def combine(d, acc):

    def comb_body(g, acc_in):
      base_i = g * (_NCORN * _LANES)
      val_d = val_vs[d]
      sdf = (val_d[pl.ds(base_i + 0 * _LANES, _LANES)]
             * w_v[d, pl.ds(base_i + 0 * _LANES, _LANES)])
      for cc in range(1, _NCORN):
        sdf = sdf + (val_d[pl.ds(base_i + cc * _LANES, _LANES)]
                     * w_v[d, pl.ds(base_i + cc * _LANES, _LANES)])
      ax = jnp.abs(sdf)
      hub = jnp.where(ax < 1.0, 0.5 * sdf * sdf, ax - 0.5)
      return acc_in + hub

    return lax.fori_loop(0, _GROUPS, comb_body, acc)

  acc = jnp.zeros((_LANES,), jnp.float32)
  for sp in range(_SPS):
    k = c * _SPS + sp
    row0 = k * 8
    for h in range(2):
      plsc.subcore_barrier()
      vb = (k * 8 + h * _BPH) * _VOX_B + s * (_SPM // _NS)
      pltpu.sync_copy(vox_hbm.at[pl.ds(vb, _SPM // _NS)],
                      sp_v.at[pl.ds(s * (_SPM // _NS), _SPM // _NS)])
      plsc.subcore_barrier()

      for dp in fire_pts(row0, 0, 0):
        dp.wait()
      compute(h, 0)
      g_desc = {0: fire_gather(0)}
      p_descs = {1: fire_pts(row0, 1, 1)}
      for u in range(1, _NSUB):
        d = u % 2
        for dp in p_descs.pop(u):
          dp.wait()
        compute(h, d)
        g_desc[u] = fire_gather(d)
        if u + 1 < _NSUB:
          p_descs[u + 1] = fire_pts(row0, u + 1, (u + 1) % 2)
        g_desc.pop(u - 1).wait()
        acc = combine(1 - d, acc)
      g_desc.pop(_NSUB - 1).wait()
      acc = combine((_NSUB - 1) % 2, acc)

  part_v[...] = acc
  pltpu.sync_copy(part_v, out_hbm.at[s * _NC + c])


@jax.jit
def kernel(voxels, pts_centroid, height_gt):
  vox_flat = voxels.transpose(0, 1, 3, 2).reshape(-1)
  xs = pts_centroid[..., 0]
  ys = pts_centroid[..., 1]
  zs = pts_centroid[..., 2]
  hs = height_gt

  mesh = plsc.VectorSubcoreMesh(
      core_axis_name="c", subcore_axis_name="s",
      num_cores=_NC, num_subcores=_NS)
  kfn = pl.kernel(
      _tec_body,
      out_type=jax.ShapeDtypeStruct((_NW, _LANES), jnp.float32),
      mesh=mesh,
      scratch_types=[
          pltpu.VMEM((2, 8, _COLS), jnp.float32),
          pltpu.VMEM((2, 8, _COLS), jnp.float32),
          pltpu.VMEM((2, 8, _COLS), jnp.float32),
          pltpu.VMEM((2, 8, _COLS), jnp.float32),
          pltpu.VMEM((_NCORN * _PPC,), jnp.int32),
          pltpu.VMEM((_NCORN * _PPC,), jnp.int32),
          pltpu.VMEM((2, _NCORN * _PPC), jnp.float32),
          pltpu.VMEM((_NCORN * _PPC,), jnp.float32),
          pltpu.VMEM((_NCORN * _PPC,), jnp.float32),
          pltpu.VMEM((_LANES,), jnp.float32),
          pltpu.VMEM_SHARED((_SPM,), jnp.float32),
          pltpu.SemaphoreType.DMA((2,)),
          pltpu.SemaphoreType.DMA((2,)),
      ],
      compiler_params=pltpu.CompilerParams(needs_layout_passes=False),
  )
  partials = kfn(vox_flat, xs, ys, zs, hs)
  return jnp.sum(partials) / _NPTS


# speedup vs baseline: 9.5670x; 1.0107x over previous
import jax
import jax.numpy as jnp
from jax import lax
from jax.experimental import pallas as pl
from jax.experimental.pallas import tpu as pltpu
from jax.experimental.pallas import tpu_sc as plsc

_NC = 2
_NS = 16
_LANES = 16
_NW = _NC * _NS

_B, _L, _W, _H = 64, 80, 80, 40
_VOX_B = _L * _W * _H
_N = 16384
_NPTS = _B * _N

_SPS = 4
_BPH = 4
_SPM = _BPH * _VOX_B
_COLS = 256
_NSUB = 4
_PPC = _BPH * _COLS
_GROUPS = _PPC // _LANES
_GPR = _COLS // _LANES
_NCORN = 8

_GRID_RES = 0.1
_INV_RES = 1.0 / _GRID_RES


def _floor_to_int(q):
  t = q.astype(jnp.int32)
  tf = t.astype(jnp.float32)
  adj = (tf > q)
  ti = jnp.where(adj, t - 1, t)
  return ti, jnp.where(adj, tf - 1.0, tf)


def _tec_body(slab, vox_hbm, xs_hbm, ys_hbm, zs_hbm, hs_hbm, out_hbm,
              x_v, y_v, z_v, h_v, idx_v0, idx_v1, w_v, val_v0, val_v1,
              part_v, sp_v, gsems, psems):
  c = lax.axis_index("c")
  s = lax.axis_index("s")
  idx_vs = (idx_v0, idx_v1)
  val_vs = (val_v0, val_v1)

  def fire_pts(row0, u, d):
    col0 = s * (_NSUB * _COLS) + u * _COLS
    return [
        pltpu.async_copy(src.at[pl.ds(row0, 8), pl.ds(col0, _COLS)],
                         dst.at[d], psems.at[d])
        for src, dst in ((xs_hbm, x_v), (ys_hbm, y_v),
                         (zs_hbm, z_v), (hs_hbm, h_v))
    ]

  def compute(hh_, d):

    def group_body(g, _):
      row = hh_ * _BPH + g // _GPR
      col = (g % _GPR) * _LANES
      lbase = (g // _GPR) * _VOX_B
      px = x_v[d, row, pl.ds(col, _LANES)]
      py = y_v[d, row, pl.ds(col, _LANES)]
      pz = z_v[d, row, pl.ds(col, _LANES)]
      hh = h_v[d, row, pl.ds(col, _LANES)]

      x = px + 4.0
      y = py + 4.0
      z = pz + hh * 0.5

      xi, xf = _floor_to_int(x * _INV_RES)
      yi, yf = _floor_to_int(y * _INV_RES)
      zi, zf = _floor_to_int(z * _INV_RES)
      tx = (x - xf * _GRID_RES) * _INV_RES
      ty = (y - yf * _GRID_RES) * _INV_RES
      tz = (z - zf * _GRID_RES) * _INV_RES

      zero = jnp.zeros((_LANES,), jnp.int32)
      xmin = jnp.clip(xi, zero, _L - 1)
      xmax = jnp.clip(xi + 1, zero, _L - 1)
      ymin = jnp.clip(yi, zero, _W - 1)
      ymax = jnp.clip(yi + 1, zero, _W - 1)
      zmin = jnp.clip(zi, zero, _H - 1)
      zmax = jnp.clip(zi + 1, zero, _H - 1)

      axmin = lbase + xmin * (_W * _H)
      axmax = lbase + xmax * (_W * _H)
      bzmin = zmin * _W
      bzmax = zmax * _W

      ux = 1.0 - tx
      uy = 1.0 - ty
      uz = 1.0 - tz
      wxy_pp = tx * ty
      wxy_pm = tx * uy
      wxy_mp = ux * ty
      wxy_mm = ux * uy

      base_i = g * (_NCORN * _LANES)
      idx_d = idx_vs[d]
      idx_d[pl.ds(base_i + 0 * _LANES, _LANES)] = axmax + bzmax + ymax
      idx_d[pl.ds(base_i + 1 * _LANES, _LANES)] = axmax + bzmin + ymax
      idx_d[pl.ds(base_i + 2 * _LANES, _LANES)] = axmax + bzmax + ymin
      idx_d[pl.ds(base_i + 3 * _LANES, _LANES)] = axmax + bzmin + ymin
      idx_d[pl.ds(base_i + 4 * _LANES, _LANES)] = axmin + bzmax + ymax
      idx_d[pl.ds(base_i + 5 * _LANES, _LANES)] = axmin + bzmin + ymax
      idx_d[pl.ds(base_i + 6 * _LANES, _LANES)] = axmin + bzmax + ymin
      idx_d[pl.ds(base_i + 7 * _LANES, _LANES)] = axmin + bzmin + ymin

      w_v[d, pl.ds(base_i + 0 * _LANES, _LANES)] = wxy_pp * tz
      w_v[d, pl.ds(base_i + 1 * _LANES, _LANES)] = wxy_pp * uz
      w_v[d, pl.ds(base_i + 2 * _LANES, _LANES)] = wxy_pm * tz
      w_v[d, pl.ds(base_i + 3 * _LANES, _LANES)] = wxy_pm * uz
      w_v[d, pl.ds(base_i + 4 * _LANES, _LANES)] = wxy_mp * tz
      w_v[d, pl.ds(base_i + 5 * _LANES, _LANES)] = wxy_mp * uz
      w_v[d, pl.ds(base_i + 6 * _LANES, _LANES)] = wxy_mm * tz
      w_v[d, pl.ds(base_i + 7 * _LANES, _LANES)] = wxy_mm * uz
      return _

    lax.fori_loop(0, _GROUPS, group_body, 0)

  def fire_gather(d):
    return pltpu.async_copy(sp_v.at[idx_vs[d]], val_vs[d], gsems.at[d])

  def combine(d, acc):

    def comb_body(g, acc_in):
      base_i = g * (_NCORN * _LANES)
      val_d = val_vs[d]
      sdf = (val_d[pl.ds(base_i + 0 * _LANES, _LANES)]
             * w_v[d, pl.ds(base_i + 0 * _LANES, _LANES)])
      for cc in range(1, _NCORN):
        sdf = sdf + (val_d[pl.ds(base_i + cc * _LANES, _LANES)]
                     * w_v[d, pl.ds(base_i + cc * _LANES, _LANES)])
      ax = jnp.abs(sdf)
      hub = jnp.where(ax < 1.0, 0.5 * sdf * sdf, ax - 0.5)
      return acc_in + hub

    return lax.fori_loop(0, _GROUPS, comb_body, acc)

  acc = jnp.zeros((_LANES,), jnp.float32)
  for sp in range(1):
    row0 = (slab * 2 + c) * 8
    for h in range(2):
      plsc.subcore_barrier()
      vb = (c * 8 + h * _BPH) * _VOX_B + s * (_SPM // _NS)
      pltpu.sync_copy(vox_hbm.at[pl.ds(vb, _SPM // _NS)],
                      sp_v.at[pl.ds(s * (_SPM // _NS), _SPM // _NS)])
      plsc.subcore_barrier()

      for dp in fire_pts(row0, 0, 0):
        dp.wait()
      compute(h, 0)
      g_desc = {0: fire_gather(0)}
      p_descs = {1: fire_pts(row0, 1, 1)}
      for u in range(1, _NSUB):
        d = u % 2
        for dp in p_descs.pop(u):
          dp.wait()
        compute(h, d)
        g_desc[u] = fire_gather(d)
        if u + 1 < _NSUB:
          p_descs[u + 1] = fire_pts(row0, u + 1, (u + 1) % 2)
        g_desc.pop(u - 1).wait()
        acc = combine(1 - d, acc)
      g_desc.pop(_NSUB - 1).wait()
      acc = combine((_NSUB - 1) % 2, acc)

  part_v[...] = acc
  pltpu.sync_copy(part_v, out_hbm.at[s * _NC + c])


@jax.jit
def kernel(voxels, pts_centroid, height_gt):
  xs = pts_centroid[..., 0]
  ys = pts_centroid[..., 1]
  zs = pts_centroid[..., 2]
  hs = height_gt

  mesh = plsc.VectorSubcoreMesh(
      core_axis_name="c", subcore_axis_name="s",
      num_cores=_NC, num_subcores=_NS)

  def make_kfn(slab):
    return pl.kernel(
        lambda *refs: _tec_body(slab, *refs),
        out_type=jax.ShapeDtypeStruct((_NW, _LANES), jnp.float32),
        mesh=mesh,
        scratch_types=[
            pltpu.VMEM((2, 8, _COLS), jnp.float32),
            pltpu.VMEM((2, 8, _COLS), jnp.float32),
            pltpu.VMEM((2, 8, _COLS), jnp.float32),
            pltpu.VMEM((2, 8, _COLS), jnp.float32),
            pltpu.VMEM((_NCORN * _PPC,), jnp.int32),
            pltpu.VMEM((_NCORN * _PPC,), jnp.int32),
            pltpu.VMEM((2, _NCORN * _PPC), jnp.float32),
            pltpu.VMEM((_NCORN * _PPC,), jnp.float32),
            pltpu.VMEM((_NCORN * _PPC,), jnp.float32),
            pltpu.VMEM((_LANES,), jnp.float32),
            pltpu.VMEM_SHARED((_SPM,), jnp.float32),
            pltpu.SemaphoreType.DMA((2,)),
            pltpu.SemaphoreType.DMA((2,)),
        ],
        compiler_params=pltpu.CompilerParams(needs_layout_passes=False),
    )

  total = jnp.zeros((), jnp.float32)
  for j in range(4):
    vox_j = lax.slice_in_dim(voxels, 16 * j, 16 * (j + 1), axis=0)
    vox_j = vox_j.transpose(0, 1, 3, 2).reshape(-1)
    total = total + jnp.sum(make_kfn(j)(vox_j, xs, ys, zs, hs))
  return total / _NPTS


# speedup vs baseline: 9.7179x; 1.0158x over previous
import jax
import jax.numpy as jnp
from jax import lax
from jax.experimental import pallas as pl
from jax.experimental.pallas import tpu as pltpu
from jax.experimental.pallas import tpu_sc as plsc

_NC = 2
_NS = 16
_LANES = 16
_NW = _NC * _NS

_B, _L, _W, _H = 64, 80, 80, 40
_VOX_B = _L * _W * _H
_N = 16384
_NPTS = _B * _N

_SPS = 4
_BPH = 4
_SPM = _BPH * _VOX_B
_COLS = 256
_NSUB = 4
_PPC = _BPH * _COLS
_GROUPS = _PPC // _LANES
_GPR = _COLS // _LANES
_NCORN = 8

_GRID_RES = 0.1
_INV_RES = 1.0 / _GRID_RES


def _floor_to_int(q):
  t = q.astype(jnp.int32)
  tf = t.astype(jnp.float32)
  adj = (tf > q)
  ti = jnp.where(adj, t - 1, t)
  return ti, jnp.where(adj, tf - 1.0, tf)


def _tec_body(slab, vox_hbm, pts_hbm, hs_hbm, out_hbm,
              x_v, y_v, z_v, h_v, idx_v0, idx_v1, w_v, val_v0, val_v1,
              part_v, sp_v, gsems, psems):
  c = lax.axis_index("c")
  s = lax.axis_index("s")
  idx_vs = (idx_v0, idx_v1)
  val_vs = (val_v0, val_v1)

  def fire_pts(row0, u, d):
    col0 = s * (_NSUB * _COLS) + u * _COLS
    ds_r, ds_c = pl.ds(row0, 8), pl.ds(col0, _COLS)
    return [
        pltpu.async_copy(pts_hbm.at[0, ds_r, ds_c], x_v.at[d], psems.at[d]),
        pltpu.async_copy(pts_hbm.at[1, ds_r, ds_c], y_v.at[d], psems.at[d]),
        pltpu.async_copy(pts_hbm.at[2, ds_r, ds_c], z_v.at[d], psems.at[d]),
        pltpu.async_copy(hs_hbm.at[ds_r, ds_c], h_v.at[d], psems.at[d]),
    ]

  def compute(hh_, d):

    def group_body(g, _):
      row = hh_ * _BPH + g // _GPR
      col = (g % _GPR) * _LANES
      lbase = (g // _GPR) * _VOX_B
      px = x_v[d, row, pl.ds(col, _LANES)]
      py = y_v[d, row, pl.ds(col, _LANES)]
      pz = z_v[d, row, pl.ds(col, _LANES)]
      hh = h_v[d, row, pl.ds(col, _LANES)]

      x = px + 4.0
      y = py + 4.0
      z = pz + hh * 0.5

      xi, xf = _floor_to_int(x * _INV_RES)
      yi, yf = _floor_to_int(y * _INV_RES)
      zi, zf = _floor_to_int(z * _INV_RES)
      tx = (x - xf * _GRID_RES) * _INV_RES
      ty = (y - yf * _GRID_RES) * _INV_RES
      tz = (z - zf * _GRID_RES) * _INV_RES

      zero = jnp.zeros((_LANES,), jnp.int32)
      xmin = jnp.clip(xi, zero, _L - 1)
      xmax = jnp.clip(xi + 1, zero, _L - 1)
      ymin = jnp.clip(yi, zero, _W - 1)
      ymax = jnp.clip(yi + 1, zero, _W - 1)
      zmin = jnp.clip(zi, zero, _H - 1)
      zmax = jnp.clip(zi + 1, zero, _H - 1)

      axmin = lbase + xmin * (_W * _H)
      axmax = lbase + xmax * (_W * _H)
      bzmin = zmin * _W
      bzmax = zmax * _W

      ux = 1.0 - tx
      uy = 1.0 - ty
      uz = 1.0 - tz
      wxy_pp = tx * ty
      wxy_pm = tx * uy
      wxy_mp = ux * ty
      wxy_mm = ux * uy

      base_i = g * (_NCORN * _LANES)
      idx_d = idx_vs[d]
      idx_d[pl.ds(base_i + 0 * _LANES, _LANES)] = axmax + bzmax + ymax
      idx_d[pl.ds(base_i + 1 * _LANES, _LANES)] = axmax + bzmin + ymax
      idx_d[pl.ds(base_i + 2 * _LANES, _LANES)] = axmax + bzmax + ymin
      idx_d[pl.ds(base_i + 3 * _LANES, _LANES)] = axmax + bzmin + ymin
      idx_d[pl.ds(base_i + 4 * _LANES, _LANES)] = axmin + bzmax + ymax
      idx_d[pl.ds(base_i + 5 * _LANES, _LANES)] = axmin + bzmin + ymax
      idx_d[pl.ds(base_i + 6 * _LANES, _LANES)] = axmin + bzmax + ymin
      idx_d[pl.ds(base_i + 7 * _LANES, _LANES)] = axmin + bzmin + ymin

      w_v[d, pl.ds(base_i + 0 * _LANES, _LANES)] = wxy_pp * tz
      w_v[d, pl.ds(base_i + 1 * _LANES, _LANES)] = wxy_pp * uz
      w_v[d, pl.ds(base_i + 2 * _LANES, _LANES)] = wxy_pm * tz
      w_v[d, pl.ds(base_i + 3 * _LANES, _LANES)] = wxy_pm * uz
      w_v[d, pl.ds(base_i + 4 * _LANES, _LANES)] = wxy_mp * tz
      w_v[d, pl.ds(base_i + 5 * _LANES, _LANES)] = wxy_mp * uz
      w_v[d, pl.ds(base_i + 6 * _LANES, _LANES)] = wxy_mm * tz
      w_v[d, pl.ds(base_i + 7 * _LANES, _LANES)] = wxy_mm * uz
      return _

    lax.fori_loop(0, _GROUPS, group_body, 0)

  def fire_gather(d):
    return pltpu.async_copy(sp_v.at[idx_vs[d]], val_vs[d], gsems.at[d])

  def combine(d, acc):

    def comb_body(g, acc_in):
      base_i = g * (_NCORN * _LANES)
      val_d = val_vs[d]
      sdf = (val_d[pl.ds(base_i + 0 * _LANES, _LANES)]
             * w_v[d, pl.ds(base_i + 0 * _LANES, _LANES)])
      for cc in range(1, _NCORN):
        sdf = sdf + (val_d[pl.ds(base_i + cc * _LANES, _LANES)]
                     * w_v[d, pl.ds(base_i + cc * _LANES, _LANES)])
      ax = jnp.abs(sdf)
      hub = jnp.where(ax < 1.0, 0.5 * sdf * sdf, ax - 0.5)
      return acc_in + hub

    return lax.fori_loop(0, _GROUPS, comb_body, acc)

  acc = jnp.zeros((_LANES,), jnp.float32)
  for sp in range(1):
    row0 = (slab * 2 + c) * 8
    for h in range(2):
      plsc.subcore_barrier()
      vb = (c * 8 + h * _BPH) * _VOX_B + s * (_SPM // _NS)
      pltpu.sync_copy(vox_hbm.at[pl.ds(vb, _SPM // _NS)],
                      sp_v.at[pl.ds(s * (_SPM // _NS), _SPM // _NS)])
      plsc.subcore_barrier()

      for dp in fire_pts(row0, 0, 0):
        dp.wait()
      compute(h, 0)
      g_desc = {0: fire_gather(0)}
      p_descs = {1: fire_pts(row0, 1, 1)}
      for u in range(1, _NSUB):
        d = u % 2
        for dp in p_descs.pop(u):
          dp.wait()
        compute(h, d)
        g_desc[u] = fire_gather(d)
        if u + 1 < _NSUB:
          p_descs[u + 1] = fire_pts(row0, u + 1, (u + 1) % 2)
        g_desc.pop(u - 1).wait()
        acc = combine(1 - d, acc)
      g_desc.pop(_NSUB - 1).wait()
      acc = combine((_NSUB - 1) % 2, acc)

  part_v[...] = acc
  pltpu.sync_copy(part_v, out_hbm.at[s * _NC + c])


@jax.jit
def kernel(voxels, pts_centroid, height_gt):
  pts3 = pts_centroid.transpose(2, 0, 1)
  hs = height_gt

  mesh = plsc.VectorSubcoreMesh(
      core_axis_name="c", subcore_axis_name="s",
      num_cores=_NC, num_subcores=_NS)

  def make_kfn(slab):
    return pl.kernel(
        lambda *refs: _tec_body(slab, *refs),
        out_type=jax.ShapeDtypeStruct((_NW, _LANES), jnp.float32),
        mesh=mesh,
        scratch_types=[
            pltpu.VMEM((2, 8, _COLS), jnp.float32),
            pltpu.VMEM((2, 8, _COLS), jnp.float32),
            pltpu.VMEM((2, 8, _COLS), jnp.float32),
            pltpu.VMEM((2, 8, _COLS), jnp.float32),
            pltpu.VMEM((_NCORN * _PPC,), jnp.int32),
            pltpu.VMEM((_NCORN * _PPC,), jnp.int32),
            pltpu.VMEM((2, _NCORN * _PPC), jnp.float32),
            pltpu.VMEM((_NCORN * _PPC,), jnp.float32),
            pltpu.VMEM((_NCORN * _PPC,), jnp.float32),
            pltpu.VMEM((_LANES,), jnp.float32),
            pltpu.VMEM_SHARED((_SPM,), jnp.float32),
            pltpu.SemaphoreType.DMA((2,)),
            pltpu.SemaphoreType.DMA((2,)),
        ],
        compiler_params=pltpu.CompilerParams(needs_layout_passes=False),
    )

  total = jnp.zeros((), jnp.float32)
  for j in range(4):
    vox_j = lax.slice_in_dim(voxels, 16 * j, 16 * (j + 1), axis=0)
    vox_j = vox_j.transpose(0, 1, 3, 2).reshape(-1)
    total = total + jnp.sum(make_kfn(j)(vox_j, pts3, hs))
  return total / _NPTS


# speedup vs baseline: 9.7344x; 1.0017x over previous
import jax
import jax.numpy as jnp
from jax import lax
from jax.experimental import pallas as pl
from jax.experimental.pallas import tpu as pltpu
from jax.experimental.pallas import tpu_sc as plsc

_NC = 2
_NS = 16
_LANES = 16
_NW = _NC * _NS

_B, _L, _W, _H = 64, 80, 80, 40
_VOX_B = _L * _W * _H
_N = 16384
_NPTS = _B * _N

_SPS = 4
_BPH = 4
_SPM = _BPH * _VOX_B
_COLS = 256
_NSUB = 4
_PPC = _BPH * _COLS
_GROUPS = _PPC // _LANES
_GPR = _COLS // _LANES
_NCORN = 8

_GRID_RES = 0.1
_INV_RES = 1.0 / _GRID_RES


def _floor_to_int(q):
  t = q.astype(jnp.int32)
  tf = t.astype(jnp.float32)
  adj = (tf > q)
  ti = jnp.where(adj, t - 1, t)
  return ti, jnp.where(adj, tf - 1.0, tf)


def _tec_body(slab, vox_hbm, pts_hbm, hs_hbm, out_hbm,
              x_v, y_v, z_v, h_v, idx_v0, idx_v1, w_v, val_v0, val_v1,
              part_v, sp_v, gsems, psems):
  c = lax.axis_index("c")
  s = lax.axis_index("s")
  idx_vs = (idx_v0, idx_v1)
  val_vs = (val_v0, val_v1)

  def fire_pts(row0, u, d):
    col0 = s * (_NSUB * _COLS) + u * _COLS
    ds_r, ds_c = pl.ds(row0, 8), pl.ds(col0, _COLS)
    return [
        pltpu.async_copy(pts_hbm.at[0, ds_r, ds_c], x_v.at[d], psems.at[d]),
        pltpu.async_copy(pts_hbm.at[1, ds_r, ds_c], y_v.at[d], psems.at[d]),
        pltpu.async_copy(pts_hbm.at[2, ds_r, ds_c], z_v.at[d], psems.at[d]),
        pltpu.async_copy(hs_hbm.at[ds_r, ds_c], h_v.at[d], psems.at[d]),
    ]

  def compute(hh_, d):

    def group_body(g, _):
      row = hh_ * _BPH + g // _GPR
      col = (g % _GPR) * _LANES
      lbase = (g // _GPR) * _VOX_B
      px = x_v[d, row, pl.ds(col, _LANES)]
      py = y_v[d, row, pl.ds(col, _LANES)]
      pz = z_v[d, row, pl.ds(col, _LANES)]
      hh = h_v[d, row, pl.ds(col, _LANES)]

      x = px + 4.0
      y = py + 4.0
      z = pz + hh * 0.5

      xi, xf = _floor_to_int(x * _INV_RES)
      yi, yf = _floor_to_int(y * _INV_RES)
      zi, zf = _floor_to_int(z * _INV_RES)
      tx = (x - xf * _GRID_RES) * _INV_RES
      ty = (y - yf * _GRID_RES) * _INV_RES
      tz = (z - zf * _GRID_RES) * _INV_RES

      zero = jnp.zeros((_LANES,), jnp.int32)
      xmin = jnp.clip(xi, zero, _L - 1)
      xmax = jnp.clip(xi + 1, zero, _L - 1)
      ymin = jnp.clip(yi, zero, _W - 1)
      ymax = jnp.clip(yi + 1, zero, _W - 1)
      zmin = jnp.clip(zi, zero, _H - 1)
      zmax = jnp.clip(zi + 1, zero, _H - 1)

      axmin = lbase + xmin * (_W * _H)
      axmax = lbase + xmax * (_W * _H)
      bzmin = zmin * _W
      bzmax = zmax * _W

      ux = 1.0 - tx
      uy = 1.0 - ty
      uz = 1.0 - tz
      wxy_pp = tx * ty
      wxy_pm = tx * uy
      wxy_mp = ux * ty
      wxy_mm = ux * uy

      base_i = g * (_NCORN * _LANES)
      idx_d = idx_vs[d]
      idx_d[pl.ds(base_i + 0 * _LANES, _LANES)] = axmax + bzmax + ymax
      idx_d[pl.ds(base_i + 1 * _LANES, _LANES)] = axmax + bzmin + ymax
      idx_d[pl.ds(base_i + 2 * _LANES, _LANES)] = axmax + bzmax + ymin
      idx_d[pl.ds(base_i + 3 * _LANES, _LANES)] = axmax + bzmin + ymin
      idx_d[pl.ds(base_i + 4 * _LANES, _LANES)] = axmin + bzmax + ymax
      idx_d[pl.ds(base_i + 5 * _LANES, _LANES)] = axmin + bzmin + ymax
      idx_d[pl.ds(base_i + 6 * _LANES, _LANES)] = axmin + bzmax + ymin
      idx_d[pl.ds(base_i + 7 * _LANES, _LANES)] = axmin + bzmin + ymin

      w_v[d, pl.ds(base_i + 0 * _LANES, _LANES)] = wxy_pp * tz
      w_v[d, pl.ds(base_i + 1 * _LANES, _LANES)] = wxy_pp * uz
      w_v[d, pl.ds(base_i + 2 * _LANES, _LANES)] = wxy_pm * tz
      w_v[d, pl.ds(base_i + 3 * _LANES, _LANES)] = wxy_pm * uz
      w_v[d, pl.ds(base_i + 4 * _LANES, _LANES)] = wxy_mp * tz
      w_v[d, pl.ds(base_i + 5 * _LANES, _LANES)] = wxy_mp * uz
      w_v[d, pl.ds(base_i + 6 * _LANES, _LANES)] = wxy_mm * tz
      w_v[d, pl.ds(base_i + 7 * _LANES, _LANES)] = wxy_mm * uz
      return _

    lax.fori_loop(0, _GROUPS, group_body, 0)

  def fire_gather(d):
    return pltpu.async_copy(sp_v.at[idx_vs[d]], val_vs[d], gsems.at[d])

  def combine(d, acc):

    def comb_body(g, acc_in):
      base_i = g * (_NCORN * _LANES)
      val_d = val_vs[d]
      sdf = (val_d[pl.ds(base_i + 0 * _LANES, _LANES)]
             * w_v[d, pl.ds(base_i + 0 * _LANES, _LANES)])
      for cc in range(1, _NCORN):
        sdf = sdf + (val_d[pl.ds(base_i + cc * _LANES, _LANES)]
                     * w_v[d, pl.ds(base_i + cc * _LANES, _LANES)])
      ax = jnp.abs(sdf)
      hub = jnp.where(ax < 1.0, 0.5 * sdf * sdf, ax - 0.5)
      return acc_in + hub

    return lax.fori_loop(0, _GROUPS, comb_body, acc)

  acc = jnp.zeros((_LANES,), jnp.float32)
  for sp in range(1):
    row0 = (slab * 2 + c) * 8
    for h in range(2):
      plsc.subcore_barrier()
      vb = (c * 8 + h * _BPH) * _VOX_B + s * (_SPM // _NS)
      pltpu.sync_copy(vox_hbm.at[pl.ds(vb, _SPM // _NS)],
                      sp_v.at[pl.ds(s * (_SPM // _NS), _SPM // _NS)])
      plsc.subcore_barrier()

      for dp in fire_pts(row0, 0, 0):
        dp.wait()
      compute(h, 0)
      g_desc = {0: fire_gather(0)}
      p_descs = {1: fire_pts(row0, 1, 1)}
      for u in range(1, _NSUB):
        d = u % 2
        for dp in p_descs.pop(u):
          dp.wait()
        compute(h, d)
        g_desc[u] = fire_gather(d)
        if u + 1 < _NSUB:
          p_descs[u + 1] = fire_pts(row0, u + 1, (u + 1) % 2)
        g_desc.pop(u - 1).wait()
        acc = combine(1 - d, acc)
      g_desc.pop(_NSUB - 1).wait()
      acc = combine((_NSUB - 1) % 2, acc)

  part_v[...] = acc
  pltpu.sync_copy(part_v, out_hbm.at[s * _NC + c])


@jax.jit
def kernel(voxels, pts_centroid, height_gt):
  pts3 = pts_centroid.transpose(2, 0, 1)
  hs = height_gt

  mesh = plsc.VectorSubcoreMesh(
      core_axis_name="c", subcore_axis_name="s",
      num_cores=_NC, num_subcores=_NS)

  def make_kfn(slab):
    return pl.kernel(
        lambda *refs: _tec_body(slab, *refs),
        out_type=jax.ShapeDtypeStruct((_NW, _LANES), jnp.float32),
        mesh=mesh,
        scratch_types=[
            pltpu.VMEM((2, 8, _COLS), jnp.float32),
            pltpu.VMEM((2, 8, _COLS), jnp.float32),
            pltpu.VMEM((2, 8, _COLS), jnp.float32),
            pltpu.VMEM((2, 8, _COLS), jnp.float32),
            pltpu.VMEM((_NCORN * _PPC,), jnp.int32),
            pltpu.VMEM((_NCORN * _PPC,), jnp.int32),
            pltpu.VMEM((2, _NCORN * _PPC), jnp.float32),
            pltpu.VMEM((_NCORN * _PPC,), jnp.float32),
            pltpu.VMEM((_NCORN * _PPC,), jnp.float32),
            pltpu.VMEM((_LANES,), jnp.float32),
            pltpu.VMEM_SHARED((_SPM,), jnp.float32),
            pltpu.SemaphoreType.DMA((2,)),
            pltpu.SemaphoreType.DMA((2,)),
        ],
        compiler_params=pltpu.CompilerParams(needs_layout_passes=False),
    )

  vox_t = voxels.transpose(0, 1, 3, 2)
  total = jnp.zeros((), jnp.float32)
  for j in range(4):
    vox_j = lax.slice_in_dim(vox_t, 16 * j, 16 * (j + 1), axis=0)
    total = total + jnp.sum(make_kfn(j)(vox_j.reshape(-1), pts3, hs))
  return total / _NPTS
